# baseline (speedup 1.0000x reference)
import functools

import jax
import jax.numpy as jnp
from jax import lax
from jax.experimental import pallas as pl
from jax.experimental.pallas import tpu as pltpu
from jax.experimental.pallas import tpu_sc as plsc

_LANES = 16
_CHUNK = 128
_SPLIT = 28


def _dense_pre(xp, p16, W1, n_pad, blk):
    def body(x_ref, p_ref, w_ref, t0_ref, t1_ref):
        a = jnp.dot(x_ref[...], w_ref[...], preferred_element_type=jnp.float32)
        t0_ref[...] = jnp.concatenate([a[:, :_SPLIT], p_ref[...]], axis=1)
        t1_ref[...] = a[:, _SPLIT:]

    return pl.pallas_call(
        body,
        grid=(n_pad // blk,),
        in_specs=[
            pl.BlockSpec((blk, xp.shape[1]), lambda i: (i, 0)),
            pl.BlockSpec((blk, 8), lambda i: (i, 0)),
            pl.BlockSpec(W1.shape, lambda i: (0, 0)),
        ],
        out_specs=[pl.BlockSpec((blk, 36), lambda i: (i, 0))] * 2,
        out_shape=[jax.ShapeDtypeStruct((n_pad, 36), jnp.float32)] * 2,
    )(xp, p16, W1)


_IB = 8
_ECH = 64
_EIB = 4


def _edge_agg(t0, t1, src, dst, n_pad, e_pad):
    rows = n_pad // 16
    chunks = e_pad // 16 // _ECH
    blocks = chunks // _EIB
    src2 = src.reshape(-1, _ECH)
    dst2 = dst.reshape(-1, _ECH)
    mesh = plsc.VectorSubcoreMesh(core_axis_name="c", subcore_axis_name="s")

    @functools.partial(
        pl.kernel,
        out_type=[jax.ShapeDtypeStruct((n_pad, 36), jnp.float32)] * 2,
        mesh=mesh,
        compiler_params=pltpu.CompilerParams(use_tc_tiling_on_sc=False),
        scratch_types=[
            pltpu.VMEM_SHARED((n_pad, 36), jnp.float32),
            pltpu.VMEM((_EIB, _ECH), jnp.int32),
            pltpu.VMEM((_EIB, _ECH), jnp.int32),
            pltpu.VMEM((_ECH, 36), jnp.float32),
            pltpu.VMEM((_ECH, 36), jnp.float32),
            pltpu.SemaphoreType.DMA,
            pltpu.SemaphoreType.DMA,
        ],
    )
    def k(t0_hbm, t1_hbm, src_hbm, dst_hbm, o0_hbm, o1_hbm,
          acc, isrc, idst, rb0, rb1, gsem, ssem):
        c = lax.axis_index("c")
        s = lax.axis_index("s")
        rbase = s * rows
        bufs = (rb0, rb1)

        def work(tbl, out):
            pltpu.sync_copy(tbl.at[pl.ds(rbase, rows)],
                            acc.at[pl.ds(rbase, rows)])
            plsc.subcore_barrier()

            def step(b, carry):
                row0 = s * chunks + b * _EIB
                pltpu.sync_copy(src_hbm.at[pl.ds(row0, _EIB)], isrc)
                pltpu.sync_copy(dst_hbm.at[pl.ds(row0, _EIB)], idst)
                gd = [None] * _EIB
                sd = [None] * _EIB
                gd[0] = pltpu.async_copy(tbl.at[isrc.at[0]], bufs[0], gsem)
                for j in range(_EIB):
                    gd[j].wait()
                    sd[j] = pltpu.async_copy(
                        bufs[j % 2], acc.at[idst.at[j]], ssem, add=True)
                    if j + 1 < _EIB:
                        if j >= 1:
                            sd[j - 1].wait()
                        gd[j + 1] = pltpu.async_copy(
                            tbl.at[isrc.at[j + 1]], bufs[(j + 1) % 2], gsem)
                sd[_EIB - 2].wait()
                sd[_EIB - 1].wait()
                return carry

            lax.fori_loop(0, blocks, step, 0)
            plsc.subcore_barrier()
            pltpu.sync_copy(acc.at[pl.ds(rbase, rows)],
                            out.at[pl.ds(rbase, rows)])

        pl.when(c == 0)(lambda: work(t0_hbm, o0_hbm))
        pl.when(c == 1)(lambda: work(t1_hbm, o1_hbm))

    return k(t0, t1, src2, dst2)


def _dense_post(g0, g1, b1, wp16, bp, W2, n_pad, blk):
    def body(g0_ref, g1_ref, b1_ref, wp_ref, bp_ref, w2_ref, u_ref):
        g0v = g0_ref[...]
        x1 = jnp.maximum(
            jnp.concatenate([g0v[:, :_SPLIT], g1_ref[...]], axis=1)
            + b1_ref[...], 0.0)
        p1 = jnp.maximum(
            jnp.dot(g0v[:, _SPLIT:], wp_ref[...],
                    preferred_element_type=jnp.float32) + bp_ref[...], 0.0)
        h = jnp.concatenate([x1, p1], axis=1)
        u = jnp.dot(h, jnp.sum(w2_ref[...], axis=1))
        u_ref[...] = jnp.broadcast_to(u[:, None], u_ref.shape)

    return pl.pallas_call(
        body,
        grid=(n_pad // blk,),
        in_specs=[
            pl.BlockSpec((blk, 36), lambda i: (i, 0)),
            pl.BlockSpec((blk, 36), lambda i: (i, 0)),
            pl.BlockSpec((1, 64), lambda i: (0, 0)),
            pl.BlockSpec((8, 64), lambda i: (0, 0)),
            pl.BlockSpec((1, 64), lambda i: (0, 0)),
            pl.BlockSpec((128, 128), lambda i: (0, 0)),
        ],
        out_specs=pl.BlockSpec((blk, 8), lambda i: (i, 0)),
        out_shape=jax.ShapeDtypeStruct((n_pad, 8), jnp.float32),
    )(g0, g1, b1[None, :], wp16, bp[None, :], W2)


def _scalar_stage(u, src, dst, b2, n_pad, e_pad):
    rows = n_pad // 16
    chunks = e_pad // 16 // _CHUNK
    blocks = chunks // _IB
    src2 = src.reshape(-1, _CHUNK)
    dst2 = dst.reshape(-1, _CHUNK)
    mesh = plsc.VectorSubcoreMesh(core_axis_name="c", subcore_axis_name="s",
                                  num_cores=1)

    @functools.partial(
        pl.kernel,
        out_type=jax.ShapeDtypeStruct((e_pad // _CHUNK, _CHUNK), jnp.float32),
        mesh=mesh,
        compiler_params=pltpu.CompilerParams(use_tc_tiling_on_sc=False,
                                             needs_layout_passes=False),
        scratch_types=[
            pltpu.VMEM_SHARED((n_pad,), jnp.float32),
            pltpu.VMEM((n_pad,), jnp.float32),
            pltpu.VMEM((_IB, _CHUNK), jnp.int32),
            pltpu.VMEM((_IB, _CHUNK), jnp.int32),
            pltpu.VMEM((_IB, _CHUNK), jnp.float32),
            pltpu.VMEM((_IB, _CHUNK), jnp.float32),
            pltpu.VMEM((_CHUNK,), jnp.float32),
            pltpu.SemaphoreType.DMA,
        ],
    )
    def k(u_hbm, src_hbm, dst_hbm, b2_hbm, out_hbm,
          acc, uloc, isrc, idst, vals, obuf, b2v, ssem):
        s = lax.axis_index("s")
        rbase = s * rows
        pltpu.sync_copy(u_hbm.at[pl.ds(rbase, rows)],
                        acc.at[pl.ds(rbase, rows)])
        pltpu.sync_copy(u_hbm, uloc)
        pltpu.sync_copy(b2_hbm, b2v)
        plsc.subcore_barrier()

        def step(b, carry):
            row0 = s * chunks + b * _IB
            pltpu.sync_copy(src_hbm.at[pl.ds(row0, _IB)], isrc)
            pltpu.sync_copy(dst_hbm.at[pl.ds(row0, _IB)], idst)
            descs = []
            for j in range(_IB):
                for i in range(_CHUNK // _LANES):
                    iv = isrc[j, pl.ds(i * _LANES, _LANES)]
                    vals[j, pl.ds(i * _LANES, _LANES)] = (
                        plsc.load_gather(uloc, [iv]))
                descs.append(pltpu.async_copy(
                    vals.at[j], acc.at[idst.at[j]], ssem, add=True))
            for d in descs:
                d.wait()
            return carry

        lax.fori_loop(0, blocks, step, 0)
        plsc.subcore_barrier()
        pltpu.sync_copy(acc, uloc)
        t16 = b2v[pl.ds(0, _LANES)]
        for i in range(1, 8):
            t16 = t16 + b2v[pl.ds(i * _LANES, _LANES)]
        b2s = jnp.sum(t16)

        def step2(b, carry):
            row0 = s * chunks + b * _IB
            pltpu.sync_copy(dst_hbm.at[pl.ds(row0, _IB)], idst)
            for j in range(_IB):
                for i in range(_CHUNK // _LANES):
                    iv = idst[j, pl.ds(i * _LANES, _LANES)]
                    obuf[j, pl.ds(i * _LANES, _LANES)] = (
                        plsc.load_gather(uloc, [iv]) + b2s)
            pltpu.sync_copy(obuf, out_hbm.at[pl.ds(row0, _IB)])
            return carry

        lax.fori_loop(0, blocks, step2, 0)

    return k(u, src2, dst2, b2)


def kernel(x, edge_index, pos_embeddings, W1, b1, Wp, bp, W2, b2):
    N = x.shape[1]
    E = edge_index.shape[1]
    xf = x.reshape(N, x.shape[2])
    pf = pos_embeddings.reshape(N, pos_embeddings.shape[2])

    n_pad = ((N + 127) // 128) * 128
    e_pad = ((E + 4095) // 4096) * 4096
    blk = n_pad // 16

    xp = jnp.pad(xf, ((0, n_pad - N), (0, 0)))
    p16 = jnp.pad(pf, ((0, n_pad - N), (0, 8 - pf.shape[1])))
    wp16 = jnp.pad(Wp, ((0, 8 - Wp.shape[0]), (0, 0)))

    src = jnp.pad(edge_index[0], (0, e_pad - E))
    pad_dst = N + (jnp.arange(e_pad - E, dtype=jnp.int32) % (n_pad - N))
    dst = jnp.concatenate([edge_index[1], pad_dst])

    t0, t1 = _dense_pre(xp, p16, W1, n_pad, blk)
    g0, g1 = _edge_agg(t0, t1, src, dst, n_pad, e_pad)
    u = _dense_post(g0, g1, b1, wp16, bp, W2, n_pad, blk)[:, 0]
    ratings = _scalar_stage(u, src, dst, b2, n_pad, e_pad)
    return ratings.reshape(e_pad)[:E]


# speedup vs baseline: 1.0279x; 1.0279x over previous
import functools

import jax
import jax.numpy as jnp
from jax import lax
from jax.experimental import pallas as pl
from jax.experimental.pallas import tpu as pltpu
from jax.experimental.pallas import tpu_sc as plsc

_LANES = 16
_CHUNK = 128
_SPLIT = 24


def _dense_pre(xp, p16, W1, n_pad, blk):
    def body(x_ref, p_ref, w_ref, t0_ref, t1_ref):
        a = jnp.dot(x_ref[...], w_ref[...], preferred_element_type=jnp.float32)
        t0_ref[...] = jnp.concatenate([a[:, :_SPLIT], p_ref[...]], axis=1)
        t1_ref[...] = a[:, _SPLIT:]

    return pl.pallas_call(
        body,
        grid=(n_pad // blk,),
        in_specs=[
            pl.BlockSpec((blk, xp.shape[1]), lambda i: (i, 0)),
            pl.BlockSpec((blk, 16), lambda i: (i, 0)),
            pl.BlockSpec(W1.shape, lambda i: (0, 0)),
        ],
        out_specs=[pl.BlockSpec((blk, 40), lambda i: (i, 0))] * 2,
        out_shape=[jax.ShapeDtypeStruct((n_pad, 40), jnp.float32)] * 2,
    )(xp, p16, W1)


_IB = 8
_ECH = 64
_EIB = 4


def _edge_agg(t0, t1, src, dst, n_pad, e_pad):
    rows = n_pad // 16
    chunks = e_pad // 16 // _ECH
    blocks = chunks // _EIB
    src2 = src.reshape(-1, _ECH)
    dst2 = dst.reshape(-1, _ECH)
    mesh = plsc.VectorSubcoreMesh(core_axis_name="c", subcore_axis_name="s")

    @functools.partial(
        pl.kernel,
        out_type=[jax.ShapeDtypeStruct((n_pad, 40), jnp.float32)] * 2,
        mesh=mesh,
        compiler_params=pltpu.CompilerParams(use_tc_tiling_on_sc=False),
        scratch_types=[
            pltpu.VMEM_SHARED((n_pad, 40), jnp.float32),
            pltpu.VMEM((_EIB, _ECH), jnp.int32),
            pltpu.VMEM((_EIB, _ECH), jnp.int32),
            pltpu.VMEM((_ECH, 40), jnp.float32),
            pltpu.VMEM((_ECH, 40), jnp.float32),
            pltpu.SemaphoreType.DMA,
            pltpu.SemaphoreType.DMA,
        ],
    )
    def k(t0_hbm, t1_hbm, src_hbm, dst_hbm, o0_hbm, o1_hbm,
          acc, isrc, idst, rb0, rb1, gsem, ssem):
        c = lax.axis_index("c")
        s = lax.axis_index("s")
        rbase = s * rows
        bufs = (rb0, rb1)

        def work(tbl, out):
            pltpu.sync_copy(tbl.at[pl.ds(rbase, rows)],
                            acc.at[pl.ds(rbase, rows)])
            plsc.subcore_barrier()

            def step(b, carry):
                row0 = s * chunks + b * _EIB
                pltpu.sync_copy(src_hbm.at[pl.ds(row0, _EIB)], isrc)
                pltpu.sync_copy(dst_hbm.at[pl.ds(row0, _EIB)], idst)
                gd = [None] * _EIB
                sd = [None] * _EIB
                gd[0] = pltpu.async_copy(tbl.at[isrc.at[0]], bufs[0], gsem)
                for j in range(_EIB):
                    gd[j].wait()
                    sd[j] = pltpu.async_copy(
                        bufs[j % 2], acc.at[idst.at[j]], ssem, add=True)
                    if j + 1 < _EIB:
                        if j >= 1:
                            sd[j - 1].wait()
                        gd[j + 1] = pltpu.async_copy(
                            tbl.at[isrc.at[j + 1]], bufs[(j + 1) % 2], gsem)
                sd[_EIB - 2].wait()
                sd[_EIB - 1].wait()
                return carry

            lax.fori_loop(0, blocks, step, 0)
            plsc.subcore_barrier()
            pltpu.sync_copy(acc.at[pl.ds(rbase, rows)],
                            out.at[pl.ds(rbase, rows)])

        pl.when(c == 0)(lambda: work(t0_hbm, o0_hbm))
        pl.when(c == 1)(lambda: work(t1_hbm, o1_hbm))

    return k(t0, t1, src2, dst2)


def _dense_post(g0, g1, b1, wp16, bp, W2, n_pad, blk):
    def body(g0_ref, g1_ref, b1_ref, wp_ref, bp_ref, w2_ref, u_ref):
        g0v = g0_ref[...]
        x1 = jnp.maximum(
            jnp.concatenate([g0v[:, :_SPLIT], g1_ref[...]], axis=1)
            + b1_ref[...], 0.0)
        p1 = jnp.maximum(
            jnp.dot(g0v[:, _SPLIT:], wp_ref[...],
                    preferred_element_type=jnp.float32) + bp_ref[...], 0.0)
        h = jnp.concatenate([x1, p1], axis=1)
        u = jnp.dot(h, jnp.sum(w2_ref[...], axis=1))
        u_ref[...] = jnp.broadcast_to(u[:, None], u_ref.shape)

    return pl.pallas_call(
        body,
        grid=(n_pad // blk,),
        in_specs=[
            pl.BlockSpec((blk, 40), lambda i: (i, 0)),
            pl.BlockSpec((blk, 40), lambda i: (i, 0)),
            pl.BlockSpec((1, 64), lambda i: (0, 0)),
            pl.BlockSpec((16, 64), lambda i: (0, 0)),
            pl.BlockSpec((1, 64), lambda i: (0, 0)),
            pl.BlockSpec((128, 128), lambda i: (0, 0)),
        ],
        out_specs=pl.BlockSpec((blk, 8), lambda i: (i, 0)),
        out_shape=jax.ShapeDtypeStruct((n_pad, 8), jnp.float32),
    )(g0, g1, b1[None, :], wp16, bp[None, :], W2)


def _scalar_stage(u, src, dst, b2, n_pad, e_pad):
    rows = n_pad // 16
    chunks = e_pad // 16 // _CHUNK
    blocks = chunks // _IB
    src2 = src.reshape(-1, _CHUNK)
    dst2 = dst.reshape(-1, _CHUNK)
    mesh = plsc.VectorSubcoreMesh(core_axis_name="c", subcore_axis_name="s",
                                  num_cores=1)

    @functools.partial(
        pl.kernel,
        out_type=jax.ShapeDtypeStruct((e_pad // _CHUNK, _CHUNK), jnp.float32),
        mesh=mesh,
        compiler_params=pltpu.CompilerParams(use_tc_tiling_on_sc=False,
                                             needs_layout_passes=False),
        scratch_types=[
            pltpu.VMEM_SHARED((n_pad,), jnp.float32),
            pltpu.VMEM((n_pad,), jnp.float32),
            pltpu.VMEM((_IB, _CHUNK), jnp.int32),
            pltpu.VMEM((_IB, _CHUNK), jnp.int32),
            pltpu.VMEM((_IB, _CHUNK), jnp.float32),
            pltpu.VMEM((_IB, _CHUNK), jnp.float32),
            pltpu.VMEM((_CHUNK,), jnp.float32),
            pltpu.SemaphoreType.DMA,
        ],
    )
    def k(u_hbm, src_hbm, dst_hbm, b2_hbm, out_hbm,
          acc, uloc, isrc, idst, vals, obuf, b2v, ssem):
        s = lax.axis_index("s")
        rbase = s * rows
        pltpu.sync_copy(u_hbm.at[pl.ds(rbase, rows)],
                        acc.at[pl.ds(rbase, rows)])
        pltpu.sync_copy(u_hbm, uloc)
        pltpu.sync_copy(b2_hbm, b2v)
        plsc.subcore_barrier()

        def step(b, carry):
            row0 = s * chunks + b * _IB
            pltpu.sync_copy(src_hbm.at[pl.ds(row0, _IB)], isrc)
            pltpu.sync_copy(dst_hbm.at[pl.ds(row0, _IB)], idst)
            descs = []
            for j in range(_IB):
                for i in range(_CHUNK // _LANES):
                    iv = isrc[j, pl.ds(i * _LANES, _LANES)]
                    vals[j, pl.ds(i * _LANES, _LANES)] = (
                        plsc.load_gather(uloc, [iv]))
                descs.append(pltpu.async_copy(
                    vals.at[j], acc.at[idst.at[j]], ssem, add=True))
            for d in descs:
                d.wait()
            return carry

        lax.fori_loop(0, blocks, step, 0)
        plsc.subcore_barrier()
        pltpu.sync_copy(acc, uloc)
        t16 = b2v[pl.ds(0, _LANES)]
        for i in range(1, 8):
            t16 = t16 + b2v[pl.ds(i * _LANES, _LANES)]
        b2s = jnp.sum(t16)

        def step2(b, carry):
            row0 = s * chunks + b * _IB
            pltpu.sync_copy(dst_hbm.at[pl.ds(row0, _IB)], idst)
            for j in range(_IB):
                for i in range(_CHUNK // _LANES):
                    iv = idst[j, pl.ds(i * _LANES, _LANES)]
                    obuf[j, pl.ds(i * _LANES, _LANES)] = (
                        plsc.load_gather(uloc, [iv]) + b2s)
            pltpu.sync_copy(obuf, out_hbm.at[pl.ds(row0, _IB)])
            return carry

        lax.fori_loop(0, blocks, step2, 0)

    return k(u, src2, dst2, b2)


def kernel(x, edge_index, pos_embeddings, W1, b1, Wp, bp, W2, b2):
    N = x.shape[1]
    E = edge_index.shape[1]
    xf = x.reshape(N, x.shape[2])
    pf = pos_embeddings.reshape(N, pos_embeddings.shape[2])

    n_pad = ((N + 127) // 128) * 128
    e_pad = ((E + 4095) // 4096) * 4096
    blk = n_pad // 16

    xp = jnp.pad(xf, ((0, n_pad - N), (0, 0)))
    p16 = jnp.pad(pf, ((0, n_pad - N), (0, 16 - pf.shape[1])))
    wp16 = jnp.pad(Wp, ((0, 16 - Wp.shape[0]), (0, 0)))

    src = jnp.pad(edge_index[0], (0, e_pad - E))
    pad_dst = N + (jnp.arange(e_pad - E, dtype=jnp.int32) % (n_pad - N))
    dst = jnp.concatenate([edge_index[1], pad_dst])

    t0, t1 = _dense_pre(xp, p16, W1, n_pad, blk)
    g0, g1 = _edge_agg(t0, t1, src, dst, n_pad, e_pad)
    u = _dense_post(g0, g1, b1, wp16, bp, W2, n_pad, blk)[:, 0]
    ratings = _scalar_stage(u, src, dst, b2, n_pad, e_pad)
    return ratings.reshape(e_pad)[:E]


# speedup vs baseline: 1.0561x; 1.0275x over previous
import functools

import jax
import jax.numpy as jnp
from jax import lax
from jax.experimental import pallas as pl
from jax.experimental.pallas import tpu as pltpu
from jax.experimental.pallas import tpu_sc as plsc

_LANES = 16
_CHUNK = 128
_SPLIT = 24


def _dense_pre(xp, p16, W1, n_pad, blk):
    def body(x_ref, p_ref, w_ref, t0_ref, t1_ref):
        a = jnp.dot(x_ref[...], w_ref[...], preferred_element_type=jnp.float32)
        t0_ref[...] = jnp.concatenate([a[:, :_SPLIT], p_ref[...]], axis=1)
        t1_ref[...] = a[:, _SPLIT:]

    return pl.pallas_call(
        body,
        grid=(n_pad // blk,),
        in_specs=[
            pl.BlockSpec((blk, xp.shape[1]), lambda i: (i, 0)),
            pl.BlockSpec((blk, 16), lambda i: (i, 0)),
            pl.BlockSpec(W1.shape, lambda i: (0, 0)),
        ],
        out_specs=[pl.BlockSpec((blk, 40), lambda i: (i, 0))] * 2,
        out_shape=[jax.ShapeDtypeStruct((n_pad, 40), jnp.float32)] * 2,
    )(xp, p16, W1)


_IB = 14
_ECH = 64
_EIB = 4


def _edge_agg(t0, t1, src, dst, n_pad, e_pad):
    rows = n_pad // 16
    chunks = e_pad // 16 // _ECH
    blocks = chunks // _EIB
    src2 = src.reshape(-1, _ECH)
    dst2 = dst.reshape(-1, _ECH)
    mesh = plsc.VectorSubcoreMesh(core_axis_name="c", subcore_axis_name="s")

    @functools.partial(
        pl.kernel,
        out_type=[jax.ShapeDtypeStruct((n_pad, 40), jnp.float32)] * 2,
        mesh=mesh,
        compiler_params=pltpu.CompilerParams(use_tc_tiling_on_sc=False),
        scratch_types=[
            pltpu.VMEM_SHARED((n_pad, 40), jnp.float32),
            pltpu.VMEM((_EIB, _ECH), jnp.int32),
            pltpu.VMEM((_EIB, _ECH), jnp.int32),
            pltpu.VMEM((_ECH, 40), jnp.float32),
            pltpu.VMEM((_ECH, 40), jnp.float32),
            pltpu.SemaphoreType.DMA,
            pltpu.SemaphoreType.DMA,
        ],
    )
    def k(t0_hbm, t1_hbm, src_hbm, dst_hbm, o0_hbm, o1_hbm,
          acc, isrc, idst, rb0, rb1, gsem, ssem):
        c = lax.axis_index("c")
        s = lax.axis_index("s")
        rbase = s * rows
        bufs = (rb0, rb1)

        def work(tbl, out):
            pltpu.sync_copy(tbl.at[pl.ds(rbase, rows)],
                            acc.at[pl.ds(rbase, rows)])
            plsc.subcore_barrier()

            def step(b, carry):
                row0 = s * chunks + b * _EIB
                pltpu.sync_copy(src_hbm.at[pl.ds(row0, _EIB)], isrc)
                pltpu.sync_copy(dst_hbm.at[pl.ds(row0, _EIB)], idst)
                gd = [None] * _EIB
                sd = [None] * _EIB
                gd[0] = pltpu.async_copy(tbl.at[isrc.at[0]], bufs[0], gsem)
                for j in range(_EIB):
                    gd[j].wait()
                    sd[j] = pltpu.async_copy(
                        bufs[j % 2], acc.at[idst.at[j]], ssem, add=True)
                    if j + 1 < _EIB:
                        if j >= 1:
                            sd[j - 1].wait()
                        gd[j + 1] = pltpu.async_copy(
                            tbl.at[isrc.at[j + 1]], bufs[(j + 1) % 2], gsem)
                sd[_EIB - 2].wait()
                sd[_EIB - 1].wait()
                return carry

            lax.fori_loop(0, blocks, step, 0)
            plsc.subcore_barrier()
            pltpu.sync_copy(acc.at[pl.ds(rbase, rows)],
                            out.at[pl.ds(rbase, rows)])

        pl.when(c == 0)(lambda: work(t0_hbm, o0_hbm))
        pl.when(c == 1)(lambda: work(t1_hbm, o1_hbm))

    return k(t0, t1, src2, dst2)


def _dense_post(g0, g1, b1, wp16, bp, W2, n_pad, blk):
    def body(g0_ref, g1_ref, b1_ref, wp_ref, bp_ref, w2_ref, u_ref):
        g0v = g0_ref[...]
        x1 = jnp.maximum(
            jnp.concatenate([g0v[:, :_SPLIT], g1_ref[...]], axis=1)
            + b1_ref[...], 0.0)
        p1 = jnp.maximum(
            jnp.dot(g0v[:, _SPLIT:], wp_ref[...],
                    preferred_element_type=jnp.float32) + bp_ref[...], 0.0)
        h = jnp.concatenate([x1, p1], axis=1)
        u = jnp.dot(h, jnp.sum(w2_ref[...], axis=1))
        u_ref[...] = jnp.broadcast_to(u[:, None], u_ref.shape)

    return pl.pallas_call(
        body,
        grid=(n_pad // blk,),
        in_specs=[
            pl.BlockSpec((blk, 40), lambda i: (i, 0)),
            pl.BlockSpec((blk, 40), lambda i: (i, 0)),
            pl.BlockSpec((1, 64), lambda i: (0, 0)),
            pl.BlockSpec((16, 64), lambda i: (0, 0)),
            pl.BlockSpec((1, 64), lambda i: (0, 0)),
            pl.BlockSpec((128, 128), lambda i: (0, 0)),
        ],
        out_specs=pl.BlockSpec((blk, 8), lambda i: (i, 0)),
        out_shape=jax.ShapeDtypeStruct((n_pad, 8), jnp.float32),
    )(g0, g1, b1[None, :], wp16, bp[None, :], W2)


def _scalar_stage(u, src, dst, b2, n_pad, e_pad):
    rows = n_pad // 16
    chunks = e_pad // 16 // _CHUNK
    blocks = chunks // _IB
    src2 = src.reshape(-1, _CHUNK)
    dst2 = dst.reshape(-1, _CHUNK)
    mesh = plsc.VectorSubcoreMesh(core_axis_name="c", subcore_axis_name="s",
                                  num_cores=1)

    @functools.partial(
        pl.kernel,
        out_type=jax.ShapeDtypeStruct((e_pad // _CHUNK, _CHUNK), jnp.float32),
        mesh=mesh,
        compiler_params=pltpu.CompilerParams(use_tc_tiling_on_sc=False,
                                             needs_layout_passes=False),
        scratch_types=[
            pltpu.VMEM_SHARED((n_pad,), jnp.float32),
            pltpu.VMEM((n_pad,), jnp.float32),
            pltpu.VMEM((_IB, _CHUNK), jnp.int32),
            pltpu.VMEM((_IB, _CHUNK), jnp.int32),
            pltpu.VMEM((_IB, _CHUNK), jnp.float32),
            pltpu.VMEM((_IB, _CHUNK), jnp.float32),
            pltpu.VMEM((_CHUNK,), jnp.float32),
            pltpu.SemaphoreType.DMA,
        ],
    )
    def k(u_hbm, src_hbm, dst_hbm, b2_hbm, out_hbm,
          acc, uloc, isrc, idst, vals, obuf, b2v, ssem):
        s = lax.axis_index("s")
        rbase = s * rows
        pltpu.sync_copy(u_hbm.at[pl.ds(rbase, rows)],
                        acc.at[pl.ds(rbase, rows)])
        pltpu.sync_copy(u_hbm, uloc)
        pltpu.sync_copy(b2_hbm, b2v)
        plsc.subcore_barrier()

        def step(b, carry):
            row0 = s * chunks + b * _IB
            pltpu.sync_copy(src_hbm.at[pl.ds(row0, _IB)], isrc)
            pltpu.sync_copy(dst_hbm.at[pl.ds(row0, _IB)], idst)
            descs = []
            for j in range(_IB):
                for i in range(_CHUNK // _LANES):
                    iv = isrc[j, pl.ds(i * _LANES, _LANES)]
                    vals[j, pl.ds(i * _LANES, _LANES)] = (
                        plsc.load_gather(uloc, [iv]))
                descs.append(pltpu.async_copy(
                    vals.at[j], acc.at[idst.at[j]], ssem, add=True))
            for d in descs:
                d.wait()
            return carry

        lax.fori_loop(0, blocks, step, 0)
        plsc.subcore_barrier()
        pltpu.sync_copy(acc, uloc)
        t16 = b2v[pl.ds(0, _LANES)]
        for i in range(1, 8):
            t16 = t16 + b2v[pl.ds(i * _LANES, _LANES)]
        b2s = jnp.sum(t16)

        def step2(b, carry):
            row0 = s * chunks + b * _IB
            pltpu.sync_copy(dst_hbm.at[pl.ds(row0, _IB)], idst)
            for j in range(_IB):
                for i in range(_CHUNK // _LANES):
                    iv = idst[j, pl.ds(i * _LANES, _LANES)]
                    obuf[j, pl.ds(i * _LANES, _LANES)] = (
                        plsc.load_gather(uloc, [iv]) + b2s)
            pltpu.sync_copy(obuf, out_hbm.at[pl.ds(row0, _IB)])
            return carry

        lax.fori_loop(0, blocks, step2, 0)

    return k(u, src2, dst2, b2)


def kernel(x, edge_index, pos_embeddings, W1, b1, Wp, bp, W2, b2):
    N = x.shape[1]
    E = edge_index.shape[1]
    xf = x.reshape(N, x.shape[2])
    pf = pos_embeddings.reshape(N, pos_embeddings.shape[2])

    n_pad = ((N + 127) // 128) * 128
    e_pad = ((E + 4095) // 4096) * 4096
    blk = n_pad // 16

    xp = jnp.pad(xf, ((0, n_pad - N), (0, 0)))
    p16 = jnp.pad(pf, ((0, n_pad - N), (0, 16 - pf.shape[1])))
    wp16 = jnp.pad(Wp, ((0, 16 - Wp.shape[0]), (0, 0)))

    src = jnp.pad(edge_index[0], (0, e_pad - E))
    pad_dst = N + (jnp.arange(e_pad - E, dtype=jnp.int32) % (n_pad - N))
    dst = jnp.concatenate([edge_index[1], pad_dst])

    t0, t1 = _dense_pre(xp, p16, W1, n_pad, blk)
    g0, g1 = _edge_agg(t0, t1, src, dst, n_pad, e_pad)
    u = _dense_post(g0, g1, b1, wp16, bp, W2, n_pad, blk)[:, 0]
    ratings = _scalar_stage(u, src, dst, b2, n_pad, e_pad)
    return ratings.reshape(e_pad)[:E]
